# Initial kernel scaffold; baseline (speedup 1.0000x reference)
import jax
import jax.numpy as jnp
from jax.experimental import pallas as pl


def kernel(data, opacity, background):
    raise NotImplementedError("write your pallas kernel here")


# speedup vs baseline: 1.3671x; 1.3671x over previous
import functools

import jax
import jax.numpy as jnp
import numpy as np
from jax.experimental import pallas as pl
from jax.experimental.pallas import tpu as pltpu

H = 128
W = 128
NG = 1024
RB = 8
NC = 512
NROWBLK = H // RB
NCHUNK = NG // NC


def _raster_kernel(dataT_ref, op_ref, out_ref):
    c = pl.program_id(2)
    j = pl.program_id(1)

    p = dataT_ref[0]
    x = jnp.tanh(p[0:1])
    y = jnp.tanh(p[1:2])
    xs = 0.5 * (x + 1.0) * W
    ys = 0.5 * (y + 1.0) * H
    sx = jnp.abs(p[2:3]) + 0.3
    sy = jnp.abs(p[3:4]) + 0.3
    theta = jax.nn.sigmoid(p[4:5]) * (2.0 * np.pi)
    cos = jnp.cos(theta)
    sin = jnp.sin(theta)
    sx2 = sx * sx
    sy2 = sy * sy
    sig_a = cos * cos * sx2 + sin * sin * sy2
    sig_b = cos * sin * (sx2 - sy2)
    sig_c = sin * sin * sx2 + cos * cos * sy2
    det = sig_a * sig_c - sig_b * sig_b
    inv_det = 1.0 / det
    ca = sig_c * inv_det
    cb = -sig_b * inv_det
    cc = sig_a * inv_det

    colop = p[5:8] * op_ref[0:1]

    xi = (jax.lax.broadcasted_iota(jnp.int32, (1, W, 1), 1)
          .astype(jnp.float32) + 0.5)
    yi = (jax.lax.broadcasted_iota(jnp.int32, (RB, 1, 1), 0)
          .astype(jnp.float32) + (j * RB + 0.5).astype(jnp.float32))
    dx = xi - xs.reshape(1, 1, NC)
    dy = yi - ys.reshape(1, 1, NC)
    tx = (-0.5 * ca.reshape(1, 1, NC)) * dx * dx
    ty = (-0.5 * cc.reshape(1, 1, NC)) * dy * dy
    dxb = cb.reshape(1, 1, NC) * dx
    power = (tx + ty) - dxb * dy
    alpha = jnp.exp(power).reshape(RB * W, NC)

    colop8 = jnp.concatenate(
        [colop, jnp.zeros((5, NC), jnp.float32)], axis=0)
    res = jax.lax.dot_general(
        alpha, colop8.T, (((1,), (0,)), ((), ())),
        preferred_element_type=jnp.float32)
    contrib = res.T.reshape(8, RB, W)

    @pl.when(c == 0)
    def _():
        out_ref[...] = jnp.zeros_like(out_ref)
    out_ref[0] += contrib


@functools.partial(jax.jit, static_argnames=())
def kernel(data, opacity, background):
    bsz = data.shape[0]
    dataT = data.transpose(0, 2, 1)
    opT = opacity.reshape(1, NG)

    out_pal = pl.pallas_call(
        _raster_kernel,
        grid=(bsz, NROWBLK, NCHUNK),
        in_specs=[
            pl.BlockSpec((1, 8, NC), lambda b, j, c: (b, 0, c)),
            pl.BlockSpec((1, NC), lambda b, j, c: (0, c)),
        ],
        out_specs=pl.BlockSpec((1, 8, RB, W), lambda b, j, c: (b, 0, j, 0)),
        out_shape=jax.ShapeDtypeStruct((bsz, 8, H, W), jnp.float32),
        compiler_params=pltpu.CompilerParams(
            dimension_semantics=("parallel", "parallel", "arbitrary")),
    )(dataT, opT)

    return out_pal[:, :3] + background[None, :, None, None]


# speedup vs baseline: 1.3741x; 1.0051x over previous
import functools

import jax
import jax.numpy as jnp
import numpy as np
from jax.experimental import pallas as pl
from jax.experimental.pallas import tpu as pltpu

H = 128
W = 128
NG = 1024
RB = 8
NC = 512
NROWBLK = H // RB
NCHUNK = NG // NC


def _prologue_kernel(dataT_ref, op_ref, drv_ref):
    p = dataT_ref[0]
    x = jnp.tanh(p[0:1])
    y = jnp.tanh(p[1:2])
    xs = 0.5 * (x + 1.0) * W
    ys = 0.5 * (y + 1.0) * H
    sx = jnp.abs(p[2:3]) + 0.3
    sy = jnp.abs(p[3:4]) + 0.3
    theta = jax.nn.sigmoid(p[4:5]) * (2.0 * np.pi)
    cos = jnp.cos(theta)
    sin = jnp.sin(theta)
    sx2 = sx * sx
    sy2 = sy * sy
    sig_a = cos * cos * sx2 + sin * sin * sy2
    sig_b = cos * sin * (sx2 - sy2)
    sig_c = sin * sin * sx2 + cos * cos * sy2
    det = sig_a * sig_c - sig_b * sig_b
    inv_det = 1.0 / det
    ca = (-0.5) * sig_c * inv_det
    cb = -sig_b * inv_det
    cc = (-0.5) * sig_a * inv_det
    colop = p[5:8] * op_ref[0:1]
    drv_ref[0] = jnp.concatenate([xs, ys, ca, cc, cb, colop], axis=0)


def _raster_kernel(drv_ref, out_ref):
    c = pl.program_id(2)
    j = pl.program_id(1)

    d = drv_ref[0]
    xs = d[0:1].reshape(1, 1, NC)
    ys = d[1:2].reshape(1, 1, NC)
    ca = d[2:3].reshape(1, 1, NC)
    cc = d[3:4].reshape(1, 1, NC)
    cb = d[4:5].reshape(1, 1, NC)

    xi = (jax.lax.broadcasted_iota(jnp.int32, (1, W, 1), 1)
          .astype(jnp.float32) + 0.5)
    yi = (jax.lax.broadcasted_iota(jnp.int32, (RB, 1, 1), 0)
          .astype(jnp.float32) + (j * RB + 0.5).astype(jnp.float32))
    dx = xi - xs
    dy = yi - ys
    tx = ca * dx * dx
    ty = cc * dy * dy
    dxb = cb * dx
    power = (tx + ty) - dxb * dy
    alpha = jnp.exp(power).astype(jnp.bfloat16).reshape(RB * W, NC)

    colop8 = jnp.concatenate(
        [d[5:8], jnp.zeros((5, NC), jnp.float32)],
        axis=0).astype(jnp.bfloat16)
    res = jax.lax.dot_general(
        alpha, colop8.T, (((1,), (0,)), ((), ())),
        preferred_element_type=jnp.float32)
    contrib = res.T.reshape(8, RB, W)

    @pl.when(c == 0)
    def _():
        out_ref[...] = jnp.zeros_like(out_ref)
    out_ref[0] += contrib


@functools.partial(jax.jit, static_argnames=())
def kernel(data, opacity, background):
    bsz = data.shape[0]
    dataT = data.transpose(0, 2, 1)
    opT = opacity.reshape(1, NG)

    derived = pl.pallas_call(
        _prologue_kernel,
        grid=(bsz,),
        in_specs=[
            pl.BlockSpec((1, 8, NG), lambda b: (b, 0, 0)),
            pl.BlockSpec((1, NG), lambda b: (0, 0)),
        ],
        out_specs=pl.BlockSpec((1, 8, NG), lambda b: (b, 0, 0)),
        out_shape=jax.ShapeDtypeStruct((bsz, 8, NG), jnp.float32),
    )(dataT, opT)

    out_pal = pl.pallas_call(
        _raster_kernel,
        grid=(bsz, NROWBLK, NCHUNK),
        in_specs=[
            pl.BlockSpec((1, 8, NC), lambda b, j, c: (b, 0, c)),
        ],
        out_specs=pl.BlockSpec((1, 8, RB, W), lambda b, j, c: (b, 0, j, 0)),
        out_shape=jax.ShapeDtypeStruct((bsz, 8, H, W), jnp.float32),
        compiler_params=pltpu.CompilerParams(
            dimension_semantics=("parallel", "parallel", "arbitrary")),
    )(derived)

    return out_pal[:, :3] + background[None, :, None, None]
